# Initial kernel scaffold; baseline (speedup 1.0000x reference)
import jax
import jax.numpy as jnp
from jax.experimental import pallas as pl


def kernel(x, weights):
    raise NotImplementedError("write your pallas kernel here")


# speedup vs baseline: 2.5753x; 2.5753x over previous
import jax
import jax.numpy as jnp
from jax.experimental import pallas as pl
from jax.experimental.pallas import tpu as pltpu

PADDING_IDX = 1
BLOCK_S = 512


def _masked_rows_kernel(x_ref, w_ref, out_ref):
    mask = (x_ref[0, 0, 0, :] != PADDING_IDX).astype(jnp.float32)
    out_ref[0, 0, :, :] = w_ref[:, :] * mask[:, None]


def kernel(x, weights):
    bsz, seq_len = x.shape
    embed_dim = weights.shape[1]
    nsb = seq_len // BLOCK_S

    w_used = jax.lax.slice(weights, (PADDING_IDX + 1, 0),
                           (PADDING_IDX + 1 + seq_len, embed_dim))
    x4 = x.reshape(bsz, nsb, 1, BLOCK_S)

    out = pl.pallas_call(
        _masked_rows_kernel,
        grid=(nsb, bsz),
        in_specs=[
            pl.BlockSpec((1, 1, 1, BLOCK_S), lambda i, j: (j, i, 0, 0)),
            pl.BlockSpec((BLOCK_S, embed_dim), lambda i, j: (i, 0)),
        ],
        out_specs=pl.BlockSpec((1, 1, BLOCK_S, embed_dim),
                               lambda i, j: (j, i, 0, 0)),
        out_shape=jax.ShapeDtypeStruct((bsz, nsb, BLOCK_S, embed_dim),
                                       jnp.float32),
        compiler_params=pltpu.CompilerParams(
            dimension_semantics=("arbitrary", "arbitrary"),
        ),
    )(x4, w_used)
    return out.reshape(bsz, seq_len, embed_dim)


# speedup vs baseline: 3.0260x; 1.1750x over previous
import jax
import jax.numpy as jnp
from jax.experimental import pallas as pl
from jax.experimental.pallas import tpu as pltpu

PADDING_IDX = 1
BLOCK_S = 1024


def _masked_rows_kernel(x_ref, w_ref, out_ref):
    mask = (x_ref[0, 0, 0, :] != PADDING_IDX).astype(jnp.float32)
    out_ref[0, 0, :, :] = w_ref[:, :] * mask[:, None]


def kernel(x, weights):
    bsz, seq_len = x.shape
    embed_dim = weights.shape[1]
    nsb = seq_len // BLOCK_S

    w_used = jax.lax.slice(weights, (PADDING_IDX + 1, 0),
                           (PADDING_IDX + 1 + seq_len, embed_dim))
    x4 = x.reshape(bsz, nsb, 1, BLOCK_S)

    out = pl.pallas_call(
        _masked_rows_kernel,
        grid=(nsb, bsz),
        in_specs=[
            pl.BlockSpec((1, 1, 1, BLOCK_S), lambda i, j: (j, i, 0, 0)),
            pl.BlockSpec((BLOCK_S, embed_dim), lambda i, j: (i, 0)),
        ],
        out_specs=pl.BlockSpec((1, 1, BLOCK_S, embed_dim),
                               lambda i, j: (j, i, 0, 0)),
        out_shape=jax.ShapeDtypeStruct((bsz, nsb, BLOCK_S, embed_dim),
                                       jnp.float32),
        compiler_params=pltpu.CompilerParams(
            dimension_semantics=("arbitrary", "arbitrary"),
        ),
    )(x4, w_used)
    return out.reshape(bsz, seq_len, embed_dim)


# speedup vs baseline: 3.2758x; 1.0825x over previous
import jax
import jax.numpy as jnp
from jax.experimental import pallas as pl
from jax.experimental.pallas import tpu as pltpu

PADDING_IDX = 1
BLOCK_S = 2048


def _masked_rows_kernel(x_ref, w_ref, out_ref):
    mask = (x_ref[0, 0, 0, :] != PADDING_IDX).astype(jnp.float32)
    out_ref[0, 0, :, :] = w_ref[:, :] * mask[:, None]


def kernel(x, weights):
    bsz, seq_len = x.shape
    embed_dim = weights.shape[1]
    nsb = seq_len // BLOCK_S

    w_used = jax.lax.slice(weights, (PADDING_IDX + 1, 0),
                           (PADDING_IDX + 1 + seq_len, embed_dim))
    x4 = x.reshape(bsz, nsb, 1, BLOCK_S)

    out = pl.pallas_call(
        _masked_rows_kernel,
        grid=(nsb, bsz),
        in_specs=[
            pl.BlockSpec((1, 1, 1, BLOCK_S), lambda i, j: (j, i, 0, 0)),
            pl.BlockSpec((BLOCK_S, embed_dim), lambda i, j: (i, 0)),
        ],
        out_specs=pl.BlockSpec((1, 1, BLOCK_S, embed_dim),
                               lambda i, j: (j, i, 0, 0)),
        out_shape=jax.ShapeDtypeStruct((bsz, nsb, BLOCK_S, embed_dim),
                                       jnp.float32),
        compiler_params=pltpu.CompilerParams(
            dimension_semantics=("arbitrary", "arbitrary"),
        ),
    )(x4, w_used)
    return out.reshape(bsz, seq_len, embed_dim)
